# baseline (device time: 92628 ns/iter reference)
import jax
import jax.numpy as jnp
from jax import lax
from jax.experimental import pallas as pl
from jax.experimental.pallas import tpu as pltpu

N_DEV = 8
BLOCK_M = 128


def kernel(x):
    m_per, n = x.shape
    n_blocks = m_per // BLOCK_M

    def body(x_ref, out_ref, carry_ref, prefix_ref, acc_ref, send_sem, recv_sem):
        b = pl.program_id(0)
        my = lax.axis_index("i")

        @pl.when(b == 0)
        def _():
            carry_ref[...] = jnp.ones((1, n), jnp.float32)

        y = x_ref[...]
        s = 1
        while s < BLOCK_M:
            pad = jnp.ones((s, n), jnp.float32)
            y = y * jnp.concatenate([pad, y[:-s, :]], axis=0)
            s *= 2
        y = y * carry_ref[...]
        out_ref[pl.ds(b * BLOCK_M, BLOCK_M), :] = y
        carry_ref[...] = y[BLOCK_M - 1 : BLOCK_M, :]

        @pl.when(b == n_blocks - 1)
        def _():
            left = my - 1
            right = my + 1

            @pl.when(my == 0)
            def _():
                prefix_ref[...] = jnp.ones((1, n), jnp.float32)

            @pl.when(my > 0)
            def _():
                recv = pltpu.make_async_remote_copy(
                    src_ref=acc_ref,
                    dst_ref=prefix_ref,
                    send_sem=send_sem,
                    recv_sem=recv_sem,
                    device_id=(left,),
                    device_id_type=pl.DeviceIdType.MESH,
                )
                recv.wait_recv()

            acc_ref[...] = prefix_ref[...] * carry_ref[...]

            @pl.when(my < N_DEV - 1)
            def _():
                send = pltpu.make_async_remote_copy(
                    src_ref=acc_ref,
                    dst_ref=prefix_ref,
                    send_sem=send_sem,
                    recv_sem=recv_sem,
                    device_id=(right,),
                    device_id_type=pl.DeviceIdType.MESH,
                )
                send.start()
                send.wait_send()

            out_ref[...] = out_ref[...] * prefix_ref[...]

    return pl.pallas_call(
        body,
        grid=(n_blocks,),
        in_specs=[pl.BlockSpec((BLOCK_M, n), lambda b: (b, 0))],
        out_specs=pl.BlockSpec((m_per, n), lambda b: (0, 0)),
        out_shape=jax.ShapeDtypeStruct((m_per, n), jnp.float32),
        scratch_shapes=[
            pltpu.VMEM((1, n), jnp.float32),
            pltpu.VMEM((1, n), jnp.float32),
            pltpu.VMEM((1, n), jnp.float32),
            pltpu.SemaphoreType.DMA,
            pltpu.SemaphoreType.DMA,
        ],
        compiler_params=pltpu.CompilerParams(
            dimension_semantics=("arbitrary",),
            vmem_limit_bytes=60 * 1024 * 1024,
        ),
    )(x)


# device time: 70345 ns/iter; 1.3168x vs baseline; 1.3168x over previous
import jax
import jax.numpy as jnp
from jax import lax
from jax.experimental import pallas as pl
from jax.experimental.pallas import tpu as pltpu

N_DEV = 8
BLOCK_M = 512


def kernel(x):
    m_per, n = x.shape
    n_blocks = m_per // BLOCK_M

    def body(
        x_ref, out_ref, carry_ref, prefix_ref, acc_ref, comm_ref, send_sems, recv_sems
    ):
        b = pl.program_id(0)
        my = lax.axis_index("i")

        @pl.when(b == 0)
        def _():
            carry_ref[...] = jnp.ones((1, n), jnp.float32)

        y = x_ref[...]
        s = 1
        while s < BLOCK_M:
            pad = jnp.ones((s, n), jnp.float32)
            y = y * jnp.concatenate([pad, y[:-s, :]], axis=0)
            s *= 2
        y = y * carry_ref[...]
        out_ref[pl.ds(b * BLOCK_M, BLOCK_M), :] = y
        carry_ref[...] = y[BLOCK_M - 1 : BLOCK_M, :]

        @pl.when(b == n_blocks - 1)
        def _():
            prefix_ref[...] = jnp.ones((1, n), jnp.float32)
            acc_ref[...] = carry_ref[...]
            for r, d in enumerate((1, 2, 4)):
                sent = my + d < N_DEV

                @pl.when(sent)
                def _():
                    send = pltpu.make_async_remote_copy(
                        src_ref=acc_ref,
                        dst_ref=comm_ref.at[r],
                        send_sem=send_sems.at[r],
                        recv_sem=recv_sems.at[r],
                        device_id=(my + d,),
                        device_id_type=pl.DeviceIdType.MESH,
                    )
                    send.start()
                    send.wait_send()

                @pl.when(my >= d)
                def _():
                    recv = pltpu.make_async_remote_copy(
                        src_ref=acc_ref,
                        dst_ref=comm_ref.at[r],
                        send_sem=send_sems.at[r],
                        recv_sem=recv_sems.at[r],
                        device_id=(my - d,),
                        device_id_type=pl.DeviceIdType.MESH,
                    )
                    recv.wait_recv()
                    prefix_ref[...] = prefix_ref[...] * comm_ref[r]
                    acc_ref[...] = acc_ref[...] * comm_ref[r]

            out_ref[...] = out_ref[...] * prefix_ref[...]

    return pl.pallas_call(
        body,
        grid=(n_blocks,),
        in_specs=[pl.BlockSpec((BLOCK_M, n), lambda b: (b, 0))],
        out_specs=pl.BlockSpec((m_per, n), lambda b: (0, 0)),
        out_shape=jax.ShapeDtypeStruct((m_per, n), jnp.float32),
        scratch_shapes=[
            pltpu.VMEM((1, n), jnp.float32),
            pltpu.VMEM((1, n), jnp.float32),
            pltpu.VMEM((1, n), jnp.float32),
            pltpu.VMEM((3, 1, n), jnp.float32),
            pltpu.SemaphoreType.DMA((3,)),
            pltpu.SemaphoreType.DMA((3,)),
        ],
        compiler_params=pltpu.CompilerParams(
            dimension_semantics=("arbitrary",),
            vmem_limit_bytes=60 * 1024 * 1024,
        ),
    )(x)
